# baseline (speedup 1.0000x reference)
import jax
import jax.numpy as jnp
from jax import lax
from jax.experimental import pallas as pl
from jax.experimental.pallas import tpu as pltpu
from jax.experimental.pallas import tpu_sc as plsc

B = 1024
S = 200
E = 64
N = B * S
EPS = 1e-3

NC = 2
NS = 16
NW = NC * NS
ROWS_PER_TILE = N // NW
CHUNK = 256
NCHUNK = ROWS_PER_TILE // CHUNK
GB = CHUNK // 16

_MAGIC = 0x5F3759DF


def _body(ids_hbm, wt_hbm, pos_hbm, gamma_hbm, beta_hbm, out_hbm,
          pos_v, g_v, b_v, ids0, ids1, sid0, sid1, rows0, rows1,
          outb0, outb1, gsem0, gsem1, ssem0, ssem1):
    ids = (ids0, ids1)
    sid = (sid0, sid1)
    rows = (rows0, rows1)
    outb = (outb0, outb1)
    gsem = (gsem0, gsem1)
    ssem = (ssem0, ssem1)
    wid = lax.axis_index("s") * NC + lax.axis_index("c")
    row0 = wid * ROWS_PER_TILE

    pltpu.sync_copy(pos_hbm.at[pl.ds(0, S * E)], pos_v)
    pltpu.sync_copy(gamma_hbm, g_v)
    pltpu.sync_copy(beta_hbm, b_v)
    g = [g_v[pl.ds(i * 16, 16)] for i in range(4)]
    bta = [b_v[pl.ds(i * 16, 16)] for i in range(4)]

    def stage(k, buf):
        base = pl.multiple_of(row0 + k * CHUNK, CHUNK)
        pltpu.sync_copy(ids_hbm.at[pl.ds(base, CHUNK)], ids[buf])
        for i in range(CHUNK // 16):
            sid[buf][pl.ds(i * 16, 16)] = lax.shift_right_logical(
                ids[buf][pl.ds(i * 16, 16)], 1)
        for off in (0, 128):
            pltpu.async_copy(
                wt_hbm.at[sid[buf].at[pl.ds(off, 128)]],
                rows[buf].at[pl.ds(off, 128)],
                gsem[buf])

    def wait_gather(buf):
        for off in (0, 128):
            pltpu.make_async_copy(
                wt_hbm.at[sid[buf].at[pl.ds(off, 128)]],
                rows[buf].at[pl.ds(off, 128)],
                gsem[buf]).wait()

    def wait_scatter(k, buf):
        pltpu.make_async_copy(
            outb[buf],
            out_hbm.at[pl.ds(pl.multiple_of((row0 + k * CHUNK) // 2, CHUNK // 2), CHUNK // 2)],
            ssem[buf]).wait()

    def compute(k, buf):
        rv = rows[buf]
        ov = outb[buf]
        cb = k * CHUNK

        def group(gi, c):
            r0 = gi * 16
            idvec = ids[buf][pl.ds(r0, 16)]
            offv = (idvec & 1) * E
            for l in range(16):
                r = r0 + l
                off_r = offv[l]
                s_r = lax.rem(cb + r, S)
                pbase = s_r * E
                t = [rv[r, pl.ds(off_r + i * 16, 16)]
                     + pos_v[pl.ds(pbase + i * 16, 16)] for i in range(4)]
                sv = (t[0] + t[1]) + (t[2] + t[3])
                sq = (t[0] * t[0] + t[1] * t[1]) + (t[2] * t[2] + t[3] * t[3])
                tot = jnp.broadcast_to(jnp.sum(sv), (16,))
                tot2 = jnp.broadcast_to(jnp.sum(sq), (16,))
                mean = tot * (1.0 / E)
                var = tot2 * (1.0 / E) - mean * mean
                x = var + EPS
                iv = jnp.int32(_MAGIC) - lax.shift_right_logical(
                    lax.bitcast_convert_type(x, jnp.int32), 1)
                y = lax.bitcast_convert_type(iv, jnp.float32)
                h = 0.5 * x
                y = y * (1.5 - h * y * y)
                y = y * (1.5 - h * y * y)
                orow = gi * 8 + (l // 2)
                ocol = (l % 2) * E
                for i in range(4):
                    a = y * g[i]
                    c0 = bta[i] - mean * a
                    ov[orow, pl.ds(ocol + i * 16, 16)] = t[i] * a + c0
            return c
        lax.fori_loop(0, GB, group, 0)

    def chunk_work(k, buf):
        @pl.when(k + 1 < NCHUNK)
        def _():
            stage(k + 1, 1 - buf)
        wait_gather(buf)

        @pl.when(k >= 2)
        def _():
            wait_scatter(k - 2, buf)
        compute(k, buf)
        pltpu.async_copy(
            outb[buf],
            out_hbm.at[pl.ds(pl.multiple_of((row0 + k * CHUNK) // 2, CHUNK // 2), CHUNK // 2)],
            ssem[buf])

    stage(0, 0)

    def two_chunks(t, c):
        chunk_work(2 * t, 0)
        chunk_work(2 * t + 1, 1)
        return c
    lax.fori_loop(0, NCHUNK // 2, two_chunks, 0)
    chunk_work(NCHUNK - 1, 0)
    wait_scatter(NCHUNK - 2, 1)
    wait_scatter(NCHUNK - 1, 0)


@jax.jit
def kernel(input_ids, word_table, pos_table, gamma, beta):
    ids_flat = input_ids.reshape(N).astype(jnp.int32)
    wt2 = word_table.reshape(500000, 128)
    pos_flat = pos_table.reshape(-1)
    mesh = plsc.VectorSubcoreMesh(core_axis_name="c", subcore_axis_name="s")
    run = pl.kernel(
        _body,
        out_type=jax.ShapeDtypeStruct((N // 2, 128), jnp.float32),
        mesh=mesh,
        scratch_types=[
            pltpu.VMEM((S * E,), jnp.float32),
            pltpu.VMEM((E,), jnp.float32),
            pltpu.VMEM((E,), jnp.float32),
            pltpu.VMEM((CHUNK,), jnp.int32),
            pltpu.VMEM((CHUNK,), jnp.int32),
            pltpu.VMEM((CHUNK,), jnp.int32),
            pltpu.VMEM((CHUNK,), jnp.int32),
            pltpu.VMEM((CHUNK, 128), jnp.float32),
            pltpu.VMEM((CHUNK, 128), jnp.float32),
            pltpu.VMEM((CHUNK // 2, 128), jnp.float32),
            pltpu.VMEM((CHUNK // 2, 128), jnp.float32),
            pltpu.SemaphoreType.DMA,
            pltpu.SemaphoreType.DMA,
            pltpu.SemaphoreType.DMA,
            pltpu.SemaphoreType.DMA,
        ],
        compiler_params=pltpu.CompilerParams(
            needs_layout_passes=False, use_tc_tiling_on_sc=True),
    )
    out = run(ids_flat, wt2, pos_flat, gamma, beta)
    return out.reshape(B, S, E)
